# Initial kernel scaffold; baseline (speedup 1.0000x reference)
import jax
import jax.numpy as jnp
from jax.experimental import pallas as pl


def kernel(x, sampled_x, features, W1, b1, W2, b2):
    raise NotImplementedError("write your pallas kernel here")


# speedup vs baseline: 14.3365x; 14.3365x over previous
import jax
import jax.numpy as jnp
from jax import lax
from jax.experimental import pallas as pl
from jax.experimental.pallas import tpu as pltpu
from jax.experimental.pallas import tpu_sc as plsc

N = 16384
M = 2048
CF = 253
CIN = 256
B = 512
NBLK = N // B
NW = 32
QCH = 32
NCH = B // QCH
GI = 3 * QCH
L = 16


def _knn_body(q_ref, st_ref, idx_ref):
    q = q_ref[...]
    st = st_ref[...]
    qq = jnp.sum(q * q, axis=1, keepdims=True)
    ss = jnp.sum(st * st, axis=0)[None, :]
    qs = lax.dot_general(q, st, (((1,), (0,)), ((), ())),
                         preferred_element_type=jnp.float32)
    d = qq + ss - 2.0 * qs
    iota = lax.broadcasted_iota(jnp.int32, (B, M), 1)
    cols = []
    for k in range(3):
        m = jnp.min(d, axis=1, keepdims=True)
        cand = jnp.where(d == m, iota, M)
        i = jnp.min(cand, axis=1, keepdims=True)
        cols.append(i)
        if k < 2:
            d = jnp.where(iota == i, jnp.float32(jnp.inf), d)
    idx_ref[0] = jnp.concatenate(cols, axis=1)


def _knn_topk(xq, st):
    return pl.pallas_call(
        _knn_body,
        grid=(NBLK,),
        in_specs=[
            pl.BlockSpec((B, 3), lambda i: (i, 0)),
            pl.BlockSpec((3, M), lambda i: (0, 0)),
        ],
        out_specs=pl.BlockSpec((1, B, 3), lambda i: (i, 0, 0)),
        out_shape=jax.ShapeDtypeStruct((NBLK, B, 3), jnp.int32),
    )(xq, st)


def _sc_body(idx_hbm, fpad_hbm, xpad_hbm, out_hbm,
             idx_v, rows, oc, xv, sem):
    cid = lax.axis_index("c")
    sid = lax.axis_index("s")
    wid = cid * 16 + sid
    base = wid * B
    pltpu.sync_copy(idx_hbm.at[wid], idx_v)
    for c in range(NCH):
        pltpu.async_copy(fpad_hbm.at[idx_v.at[c]], rows, sem).wait()
        pltpu.sync_copy(xpad_hbm.at[pl.ds(base + c * QCH, QCH)], xv)

        def row_body(r, carry):
            oc[r, pl.ds(0, L)] = (
                (rows[3 * r, pl.ds(0, L)]
                 + rows[3 * r + 1, pl.ds(0, L)]
                 + rows[3 * r + 2, pl.ds(0, L)])
                * (1.0 / 3.0) + xv[r, pl.ds(0, L)])
            for j in range(1, CIN // L):
                sl = pl.ds(j * L, L)
                oc[r, sl] = (rows[3 * r, sl] + rows[3 * r + 1, sl]
                             + rows[3 * r + 2, sl]) * (1.0 / 3.0)
            return carry

        lax.fori_loop(0, QCH, row_body, 0)
        pltpu.sync_copy(oc, out_hbm.at[pl.ds(base + c * QCH, QCH)])


def _sc_gather(idx, fpad, xpad):
    mesh = plsc.VectorSubcoreMesh(core_axis_name="c", subcore_axis_name="s")
    return pl.kernel(
        _sc_body,
        out_type=jax.ShapeDtypeStruct((N, CIN), jnp.float32),
        mesh=mesh,
        scratch_types=[
            pltpu.VMEM((NCH, GI), jnp.int32),
            pltpu.VMEM((GI, CIN), jnp.float32),
            pltpu.VMEM((QCH, CIN), jnp.float32),
            pltpu.VMEM((QCH, L), jnp.float32),
            pltpu.SemaphoreType.DMA,
        ],
    )(idx, fpad, xpad)


def _mlp_body(h_ref, w1_ref, b1_ref, w2_ref, b2_ref, o_ref):
    h = h_ref[...]
    a = lax.dot_general(h, w1_ref[...], (((1,), (1,)), ((), ())),
                        preferred_element_type=jnp.float32) + b1_ref[...]
    a = jnp.maximum(a, 0.0)
    o_ref[...] = lax.dot_general(a, w2_ref[...], (((1,), (1,)), ((), ())),
                                 preferred_element_type=jnp.float32) + b2_ref[...]


def _mlp(h, W1, b1, W2, b2):
    return pl.pallas_call(
        _mlp_body,
        grid=(NBLK,),
        in_specs=[
            pl.BlockSpec((B, CIN), lambda i: (i, 0)),
            pl.BlockSpec((CIN, CIN), lambda i: (0, 0)),
            pl.BlockSpec((1, CIN), lambda i: (0, 0)),
            pl.BlockSpec((CIN, CIN), lambda i: (0, 0)),
            pl.BlockSpec((1, CIN), lambda i: (0, 0)),
        ],
        out_specs=pl.BlockSpec((B, CIN), lambda i: (i, 0)),
        out_shape=jax.ShapeDtypeStruct((N, CIN), jnp.float32),
    )(h, W1, b1, W2, b2)


def kernel(x, sampled_x, features, W1, b1, W2, b2):
    xq = x[0]
    st = sampled_x[0].T
    fpad = jnp.pad(features[0], ((0, 0), (3, 0)))
    xpad = jnp.pad(xq, ((0, 0), (0, L - 3)))
    idx = _knn_topk(xq, st)
    h = _sc_gather(idx.reshape(NBLK, NCH, GI), fpad, xpad)
    out = _mlp(h, W1, b1.reshape(1, CIN), W2, b2.reshape(1, CIN))
    return out[None]
